# baseline (speedup 1.0000x reference)
import functools

import jax
import jax.numpy as jnp
from jax.experimental import pallas as pl
from jax.experimental.pallas import tpu as pltpu

_CHUNKS = 4


def _body(row_ref, col_ref, o_hbm, plane, sems, *, b, h, w, d):
    col = col_ref[0:w, :]
    row = row_ref[0:h, :]
    hc = h // _CHUNKS
    for q in range(_CHUNKS):
        lo = q * hc
        plane[lo:lo + hc, :, 0:d] = jnp.broadcast_to(
            col[None, :, :], (hc, w, d))
        plane[lo:lo + hc, :, d:2 * d] = jnp.broadcast_to(
            row[lo:lo + hc, None, :], (hc, w, d))
        for i in range(b):
            pltpu.make_async_copy(
                plane.at[lo:lo + hc], o_hbm.at[i, lo:lo + hc],
                sems.at[q, i]).start()
    for q in range(_CHUNKS):
        lo = q * hc
        for i in range(b):
            pltpu.make_async_copy(
                plane.at[lo:lo + hc], o_hbm.at[i, lo:lo + hc],
                sems.at[q, i]).wait()


def kernel(x, row_embed, col_embed):
    b = x.shape[0]
    h, w = x.shape[-2], x.shape[-1]
    d = row_embed.shape[1]
    body = functools.partial(_body, b=b, h=h, w=w, d=d)
    out = pl.pallas_call(
        body,
        in_specs=[
            pl.BlockSpec(memory_space=pltpu.MemorySpace.VMEM),
            pl.BlockSpec(memory_space=pltpu.MemorySpace.VMEM),
        ],
        out_specs=pl.BlockSpec(memory_space=pltpu.MemorySpace.HBM),
        out_shape=jax.ShapeDtypeStruct((b, h, w, 2 * d), jnp.float32),
        scratch_shapes=[
            pltpu.VMEM((h, w, 2 * d), jnp.float32),
            pltpu.SemaphoreType.DMA((_CHUNKS, b)),
        ],
    )(row_embed, col_embed)
    return jnp.transpose(out, (0, 3, 1, 2))


# speedup vs baseline: 1.0086x; 1.0086x over previous
import functools

import jax
import jax.numpy as jnp
from jax.experimental import pallas as pl
from jax.experimental.pallas import tpu as pltpu


def _body(row_ref, col_ref, o_hbm, plane, sems, *, b, h, w, d):
    col = col_ref[0:w, :]
    row = row_ref[0:h, :]
    h2 = h // 2
    plane[0:h2, :, 0:d] = jnp.broadcast_to(col[None, :, :], (h2, w, d))
    plane[0:h2, :, d:2 * d] = jnp.broadcast_to(
        row[0:h2, None, :], (h2, w, d))
    for i in range(b):
        pltpu.make_async_copy(
            plane.at[0:h2], o_hbm.at[i, 0:h2], sems.at[0, i]).start()
    plane[h2:h, :, 0:d] = jnp.broadcast_to(col[None, :, :], (h - h2, w, d))
    plane[h2:h, :, d:2 * d] = jnp.broadcast_to(
        row[h2:h, None, :], (h - h2, w, d))
    for i in range(b):
        pltpu.make_async_copy(
            plane.at[h2:h], o_hbm.at[i, h2:h], sems.at[1, i]).start()
    for i in range(b):
        pltpu.make_async_copy(
            plane.at[0:h2], o_hbm.at[i, 0:h2], sems.at[0, i]).wait()
    for i in range(b):
        pltpu.make_async_copy(
            plane.at[h2:h], o_hbm.at[i, h2:h], sems.at[1, i]).wait()


def kernel(x, row_embed, col_embed):
    b = x.shape[0]
    h, w = x.shape[-2], x.shape[-1]
    d = row_embed.shape[1]
    body = functools.partial(_body, b=b, h=h, w=w, d=d)
    out = pl.pallas_call(
        body,
        in_specs=[
            pl.BlockSpec(memory_space=pltpu.MemorySpace.VMEM),
            pl.BlockSpec(memory_space=pltpu.MemorySpace.VMEM),
        ],
        out_specs=pl.BlockSpec(memory_space=pltpu.MemorySpace.HBM),
        out_shape=jax.ShapeDtypeStruct((b, h, w, 2 * d), jnp.float32),
        scratch_shapes=[
            pltpu.VMEM((h, w, 2 * d), jnp.float32),
            pltpu.SemaphoreType.DMA((2, b)),
        ],
    )(row_embed, col_embed)
    return jnp.transpose(out, (0, 3, 1, 2))
